# baseline (speedup 1.0000x reference)
import jax
import jax.numpy as jnp
from jax import lax
from jax.experimental import pallas as pl
from jax.experimental.pallas import tpu as pltpu
from jax.experimental.pallas import tpu_sc as plsc

BATCH = 16384
DIM = 32

NC, NS = 2, 16
NW = NC * NS
CHUNK = BATCH // NW
NGRP = CHUNK // 16
NBUF = 2


def _sc_body(user_ref, item_ref, gu_t, gi_t, mu_t, mi_t,
             muo, mio, guvo,
             ivu, ivi, fidx, wb, stmu, stmi, stguv, sem):
  c = lax.axis_index("c")
  s = lax.axis_index("s")
  wid = s * NC + c
  base = wid * CHUNK
  pltpu.sync_copy(user_ref.at[pl.ds(base, CHUNK)], ivu)
  pltpu.sync_copy(item_ref.at[pl.ds(base, CHUNK)], ivi)
  iota = lax.iota(jnp.int32, 16)
  fidx[pl.ds(0, 16)] = iota
  fidx[pl.ds(16, 16)] = iota + 16

  tabs = (gu_t, gi_t, mu_t, mi_t)

  def group(g, carry):
    uvec = ivu[pl.ds(g * 16, 16)]
    ivec = ivi[pl.ds(g * 16, 16)]

    def issue(si):
      ru = uvec[si]
      ri = ivec[si]
      slot = si % NBUF
      cps = []
      cols = []
      for t in range(4):
        r = ru if t in (0, 2) else ri
        start = (r // 128) * 128
        cols.append(r - start)
        cps.append(pltpu.async_copy(
            tabs[t].at[fidx, pl.ds(start, 128)], wb.at[slot * 4 + t], sem))
      return cps, cols

    def extract(si, cols):
      slot = si % NBUF
      pos = jnp.full((16,), g * 16 + si, jnp.int32)
      for h in range(2):
        ridx = iota + 16 * h
        cu = jnp.full((16,), cols[0], jnp.int32)
        ci = jnp.full((16,), cols[1], jnp.int32)
        vgu = plsc.load_gather(wb.at[slot * 4 + 0], [ridx, cu])
        vgi = plsc.load_gather(wb.at[slot * 4 + 1], [ridx, ci])
        vmu = plsc.load_gather(wb.at[slot * 4 + 2], [ridx, cu])
        vmi = plsc.load_gather(wb.at[slot * 4 + 3], [ridx, ci])
        plsc.store_scatter(stguv, [ridx, pos], vgu * vgi)
        plsc.store_scatter(stmu, [ridx, pos], vmu)
        plsc.store_scatter(stmi, [ridx, pos], vmi)

    pend = [None] * NBUF
    for si in range(16):
      if pend[si % NBUF] is not None:
        pcps, pcols, psi = pend[si % NBUF]
        for cp in pcps:
          cp.wait()
        extract(psi, pcols)
      cps, cols = issue(si)
      pend[si % NBUF] = (cps, cols, si)
    for k in range(NBUF):
      pcps, pcols, psi = pend[(16 + k) % NBUF]
      for cp in pcps:
        cp.wait()
      extract(psi, pcols)
    return carry

  lax.fori_loop(0, NGRP, group, 0)

  lane = pl.ds(base, CHUNK)
  pltpu.sync_copy(stmu, muo.at[:, lane])
  pltpu.sync_copy(stmi, mio.at[:, lane])
  pltpu.sync_copy(stguv, guvo.at[:, lane])


def _sc_gather(user, item, gu_t, gi_t, mu_t, mi_t):
  mesh = plsc.VectorSubcoreMesh(core_axis_name="c", subcore_axis_name="s",
                                num_cores=NC, num_subcores=NS)
  f = pl.kernel(
      _sc_body,
      out_type=[
          jax.ShapeDtypeStruct((DIM, BATCH), jnp.float32),
          jax.ShapeDtypeStruct((DIM, BATCH), jnp.float32),
          jax.ShapeDtypeStruct((DIM, BATCH), jnp.float32),
      ],
      mesh=mesh,
      scratch_types=[
          pltpu.VMEM((CHUNK,), jnp.int32),
          pltpu.VMEM((CHUNK,), jnp.int32),
          pltpu.VMEM((DIM,), jnp.int32),
          pltpu.VMEM((4 * NBUF, DIM, 128), jnp.float32),
          pltpu.VMEM((DIM, CHUNK), jnp.float32),
          pltpu.VMEM((DIM, CHUNK), jnp.float32),
          pltpu.VMEM((DIM, CHUNK), jnp.float32),
          pltpu.SemaphoreType.DMA,
      ],
      compiler_params=pltpu.CompilerParams(needs_layout_passes=False),
  )
  return f(user, item, gu_t, gi_t, mu_t, mi_t)


def _tc_body(mu_ref, mi_ref, guv_ref, w1_ref, b1_ref, w2_ref, b2_ref,
             w3_ref, b3_ref, wo_ref, bo_ref, out_ref):
  h = jnp.concatenate([mu_ref[...], mi_ref[...]], axis=0)
  dn = (((1,), (0,)), ((), ()))
  h = jnp.maximum(
      lax.dot_general(w1_ref[...], h, dn,
                      preferred_element_type=jnp.float32) + b1_ref[...], 0.0)
  h = jnp.maximum(
      lax.dot_general(w2_ref[...], h, dn,
                      preferred_element_type=jnp.float32) + b2_ref[...], 0.0)
  h = jnp.maximum(
      lax.dot_general(w3_ref[...], h, dn,
                      preferred_element_type=jnp.float32) + b3_ref[...], 0.0)
  wo = wo_ref[...]
  dot = lax.dot_general(wo[:, :DIM], guv_ref[...], dn,
                        preferred_element_type=jnp.float32)
  dot = dot + lax.dot_general(wo[:, DIM:], h, dn,
                              preferred_element_type=jnp.float32)
  out_ref[...] = dot + bo_ref[0, 0]


def _tc_mlp(mu, mi, guv, w1, b1, w2, b2, w3, b3, wo, bo):
  nblk = 8
  blk = BATCH // nblk
  data_spec = pl.BlockSpec((DIM, blk), lambda i: (0, i))
  full = lambda shape: pl.BlockSpec(shape, lambda i: (0, 0))
  return pl.pallas_call(
      _tc_body,
      grid=(nblk,),
      in_specs=[
          data_spec, data_spec, data_spec,
          full(w1.shape), full(b1.shape),
          full(w2.shape), full(b2.shape),
          full(w3.shape), full(b3.shape),
          full(wo.shape), full(bo.shape),
      ],
      out_specs=pl.BlockSpec((1, blk), lambda i: (0, i)),
      out_shape=jax.ShapeDtypeStruct((1, BATCH), jnp.float32),
  )(mu, mi, guv, w1, b1, w2, b2, w3, b3, wo, bo)


@jax.jit
def kernel(user, item, GMF_U, GMF_I, MLP_U, MLP_I,
           W1, b1, W2, b2, W3, b3, Wo, bo):
  mu, mi, guv = _sc_gather(user, item, GMF_U.T, GMF_I.T, MLP_U.T, MLP_I.T)
  out = _tc_mlp(mu, mi, guv,
                W1, b1.reshape(-1, 1), W2, b2.reshape(-1, 1),
                W3, b3.reshape(-1, 1), Wo, bo.reshape(1, 1))
  return out.reshape(-1)


# speedup vs baseline: 1.0993x; 1.0993x over previous
import jax
import jax.numpy as jnp
from jax import lax
from jax.experimental import pallas as pl
from jax.experimental.pallas import tpu as pltpu
from jax.experimental.pallas import tpu_sc as plsc

BATCH = 16384
DIM = 32

NC, NS = 2, 16
NW = NC * NS
CHUNK = BATCH // NW
NGRP = CHUNK // 16
NBUF = 4


def _sc_body(user_ref, item_ref, gu_t, gi_t, mu_t, mi_t,
             muo, mio, guvo,
             ivu, ivi, fidx, wb, stmu, stmi, stguv, sem):
  c = lax.axis_index("c")
  s = lax.axis_index("s")
  wid = s * NC + c
  base = wid * CHUNK
  pltpu.sync_copy(user_ref.at[pl.ds(base, CHUNK)], ivu)
  pltpu.sync_copy(item_ref.at[pl.ds(base, CHUNK)], ivi)
  iota = lax.iota(jnp.int32, 16)
  fidx[pl.ds(0, 16)] = iota
  fidx[pl.ds(16, 16)] = iota + 16

  tabs = (gu_t, gi_t, mu_t, mi_t)

  def group(g, carry):
    uvec = ivu[pl.ds(g * 16, 16)]
    ivec = ivi[pl.ds(g * 16, 16)]

    def issue(si):
      ru = uvec[si]
      ri = ivec[si]
      slot = si % NBUF
      cps = []
      cols = []
      for t in range(4):
        r = ru if t in (0, 2) else ri
        start = (r // 128) * 128
        cols.append(r - start)
        cps.append(pltpu.async_copy(
            tabs[t].at[fidx, pl.ds(start, 128)], wb.at[slot * 4 + t], sem))
      return cps, cols

    def extract(si, cols):
      slot = si % NBUF
      pos = jnp.full((16,), g * 16 + si, jnp.int32)
      for h in range(2):
        ridx = iota + 16 * h
        cu = jnp.full((16,), cols[0], jnp.int32)
        ci = jnp.full((16,), cols[1], jnp.int32)
        vgu = plsc.load_gather(wb.at[slot * 4 + 0], [ridx, cu])
        vgi = plsc.load_gather(wb.at[slot * 4 + 1], [ridx, ci])
        vmu = plsc.load_gather(wb.at[slot * 4 + 2], [ridx, cu])
        vmi = plsc.load_gather(wb.at[slot * 4 + 3], [ridx, ci])
        plsc.store_scatter(stguv, [ridx, pos], vgu * vgi)
        plsc.store_scatter(stmu, [ridx, pos], vmu)
        plsc.store_scatter(stmi, [ridx, pos], vmi)

    pend = [None] * NBUF
    for si in range(16):
      if pend[si % NBUF] is not None:
        pcps, pcols, psi = pend[si % NBUF]
        for cp in pcps:
          cp.wait()
        extract(psi, pcols)
      cps, cols = issue(si)
      pend[si % NBUF] = (cps, cols, si)
    for k in range(NBUF):
      pcps, pcols, psi = pend[(16 + k) % NBUF]
      for cp in pcps:
        cp.wait()
      extract(psi, pcols)
    return carry

  lax.fori_loop(0, NGRP, group, 0)

  lane = pl.ds(base, CHUNK)
  pltpu.sync_copy(stmu, muo.at[:, lane])
  pltpu.sync_copy(stmi, mio.at[:, lane])
  pltpu.sync_copy(stguv, guvo.at[:, lane])


def _sc_gather(user, item, gu_t, gi_t, mu_t, mi_t):
  mesh = plsc.VectorSubcoreMesh(core_axis_name="c", subcore_axis_name="s",
                                num_cores=NC, num_subcores=NS)
  f = pl.kernel(
      _sc_body,
      out_type=[
          jax.ShapeDtypeStruct((DIM, BATCH), jnp.float32),
          jax.ShapeDtypeStruct((DIM, BATCH), jnp.float32),
          jax.ShapeDtypeStruct((DIM, BATCH), jnp.float32),
      ],
      mesh=mesh,
      scratch_types=[
          pltpu.VMEM((CHUNK,), jnp.int32),
          pltpu.VMEM((CHUNK,), jnp.int32),
          pltpu.VMEM((DIM,), jnp.int32),
          pltpu.VMEM((4 * NBUF, DIM, 128), jnp.float32),
          pltpu.VMEM((DIM, CHUNK), jnp.float32),
          pltpu.VMEM((DIM, CHUNK), jnp.float32),
          pltpu.VMEM((DIM, CHUNK), jnp.float32),
          pltpu.SemaphoreType.DMA,
      ],
      compiler_params=pltpu.CompilerParams(needs_layout_passes=False),
  )
  return f(user, item, gu_t, gi_t, mu_t, mi_t)


def _tc_body(mu_ref, mi_ref, guv_ref, w1_ref, b1_ref, w2_ref, b2_ref,
             w3_ref, b3_ref, wo_ref, bo_ref, out_ref):
  h = jnp.concatenate([mu_ref[...], mi_ref[...]], axis=0)
  dn = (((1,), (0,)), ((), ()))
  h = jnp.maximum(
      lax.dot_general(w1_ref[...], h, dn,
                      preferred_element_type=jnp.float32) + b1_ref[...], 0.0)
  h = jnp.maximum(
      lax.dot_general(w2_ref[...], h, dn,
                      preferred_element_type=jnp.float32) + b2_ref[...], 0.0)
  h = jnp.maximum(
      lax.dot_general(w3_ref[...], h, dn,
                      preferred_element_type=jnp.float32) + b3_ref[...], 0.0)
  wo = wo_ref[...]
  dot = lax.dot_general(wo[:, :DIM], guv_ref[...], dn,
                        preferred_element_type=jnp.float32)
  dot = dot + lax.dot_general(wo[:, DIM:], h, dn,
                              preferred_element_type=jnp.float32)
  out_ref[...] = dot + bo_ref[0, 0]


def _tc_mlp(mu, mi, guv, w1, b1, w2, b2, w3, b3, wo, bo):
  nblk = 8
  blk = BATCH // nblk
  data_spec = pl.BlockSpec((DIM, blk), lambda i: (0, i))
  full = lambda shape: pl.BlockSpec(shape, lambda i: (0, 0))
  return pl.pallas_call(
      _tc_body,
      grid=(nblk,),
      in_specs=[
          data_spec, data_spec, data_spec,
          full(w1.shape), full(b1.shape),
          full(w2.shape), full(b2.shape),
          full(w3.shape), full(b3.shape),
          full(wo.shape), full(bo.shape),
      ],
      out_specs=pl.BlockSpec((1, blk), lambda i: (0, i)),
      out_shape=jax.ShapeDtypeStruct((1, BATCH), jnp.float32),
  )(mu, mi, guv, w1, b1, w2, b2, w3, b3, wo, bo)


@jax.jit
def kernel(user, item, GMF_U, GMF_I, MLP_U, MLP_I,
           W1, b1, W2, b2, W3, b3, Wo, bo):
  mu, mi, guv = _sc_gather(user, item, GMF_U.T, GMF_I.T, MLP_U.T, MLP_I.T)
  out = _tc_mlp(mu, mi, guv,
                W1, b1.reshape(-1, 1), W2, b2.reshape(-1, 1),
                W3, b3.reshape(-1, 1), Wo, bo.reshape(1, 1))
  return out.reshape(-1)


# speedup vs baseline: 1.2868x; 1.1706x over previous
import jax
import jax.numpy as jnp
from jax import lax
from jax.experimental import pallas as pl
from jax.experimental.pallas import tpu as pltpu
from jax.experimental.pallas import tpu_sc as plsc

BATCH = 16384
DIM = 32

NC, NS = 2, 16
NW = NC * NS
CHUNK = BATCH // NW
NGRP = CHUNK // 16
NBUF = 4


def _sc_body(user_ref, item_ref, gu_t, gi_t, mu_t, mi_t,
             muo, mio, guvo,
             ivu, ivi, fidx, wb, stmu, stmi, stguv, sem):
  c = lax.axis_index("c")
  s = lax.axis_index("s")
  wid = s * NC + c
  base = wid * CHUNK
  pltpu.sync_copy(user_ref.at[pl.ds(base, CHUNK)], ivu)
  pltpu.sync_copy(item_ref.at[pl.ds(base, CHUNK)], ivi)
  iota = lax.iota(jnp.int32, 16)
  fidx[pl.ds(0, 16)] = iota
  fidx[pl.ds(16, 16)] = iota + 16

  tabs = (gu_t, gi_t, mu_t, mi_t)

  def group(g, carry):
    uvec = ivu[pl.ds(g * 16, 16)]
    ivec = ivi[pl.ds(g * 16, 16)]

    def issue(si):
      ru = uvec[si]
      ri = ivec[si]
      slot = si % NBUF
      cps = []
      cols = []
      for t in range(4):
        r = ru if t in (0, 2) else ri
        start = pl.multiple_of((r // 128) * 128, 128)
        cols.append(r - start)
        cps.append(pltpu.async_copy(
            tabs[t].at[:, pl.ds(start, 128)], wb.at[slot * 4 + t], sem))
      return cps, cols

    def extract(si, cols):
      slot = si % NBUF
      pos = jnp.full((16,), g * 16 + si, jnp.int32)
      for h in range(2):
        ridx = iota + 16 * h
        cu = jnp.full((16,), cols[0], jnp.int32)
        ci = jnp.full((16,), cols[1], jnp.int32)
        vgu = plsc.load_gather(wb.at[slot * 4 + 0], [ridx, cu])
        vgi = plsc.load_gather(wb.at[slot * 4 + 1], [ridx, ci])
        vmu = plsc.load_gather(wb.at[slot * 4 + 2], [ridx, cu])
        vmi = plsc.load_gather(wb.at[slot * 4 + 3], [ridx, ci])
        plsc.store_scatter(stguv, [ridx, pos], vgu * vgi)
        plsc.store_scatter(stmu, [ridx, pos], vmu)
        plsc.store_scatter(stmi, [ridx, pos], vmi)

    pend = [None] * NBUF
    for si in range(16):
      if pend[si % NBUF] is not None:
        pcps, pcols, psi = pend[si % NBUF]
        for cp in pcps:
          cp.wait()
        extract(psi, pcols)
      cps, cols = issue(si)
      pend[si % NBUF] = (cps, cols, si)
    for k in range(NBUF):
      pcps, pcols, psi = pend[(16 + k) % NBUF]
      for cp in pcps:
        cp.wait()
      extract(psi, pcols)
    return carry

  lax.fori_loop(0, NGRP, group, 0)

  lane = pl.ds(base, CHUNK)
  pltpu.sync_copy(stmu, muo.at[:, lane])
  pltpu.sync_copy(stmi, mio.at[:, lane])
  pltpu.sync_copy(stguv, guvo.at[:, lane])


def _sc_gather(user, item, gu_t, gi_t, mu_t, mi_t):
  mesh = plsc.VectorSubcoreMesh(core_axis_name="c", subcore_axis_name="s",
                                num_cores=NC, num_subcores=NS)
  f = pl.kernel(
      _sc_body,
      out_type=[
          jax.ShapeDtypeStruct((DIM, BATCH), jnp.float32),
          jax.ShapeDtypeStruct((DIM, BATCH), jnp.float32),
          jax.ShapeDtypeStruct((DIM, BATCH), jnp.float32),
      ],
      mesh=mesh,
      scratch_types=[
          pltpu.VMEM((CHUNK,), jnp.int32),
          pltpu.VMEM((CHUNK,), jnp.int32),
          pltpu.VMEM((DIM,), jnp.int32),
          pltpu.VMEM((4 * NBUF, DIM, 128), jnp.float32),
          pltpu.VMEM((DIM, CHUNK), jnp.float32),
          pltpu.VMEM((DIM, CHUNK), jnp.float32),
          pltpu.VMEM((DIM, CHUNK), jnp.float32),
          pltpu.SemaphoreType.DMA,
      ],
      compiler_params=pltpu.CompilerParams(needs_layout_passes=False),
  )
  return f(user, item, gu_t, gi_t, mu_t, mi_t)


def _tc_body(mu_ref, mi_ref, guv_ref, w1_ref, b1_ref, w2_ref, b2_ref,
             w3_ref, b3_ref, wo_ref, bo_ref, out_ref):
  h = jnp.concatenate([mu_ref[...], mi_ref[...]], axis=0)
  dn = (((1,), (0,)), ((), ()))
  h = jnp.maximum(
      lax.dot_general(w1_ref[...], h, dn,
                      preferred_element_type=jnp.float32) + b1_ref[...], 0.0)
  h = jnp.maximum(
      lax.dot_general(w2_ref[...], h, dn,
                      preferred_element_type=jnp.float32) + b2_ref[...], 0.0)
  h = jnp.maximum(
      lax.dot_general(w3_ref[...], h, dn,
                      preferred_element_type=jnp.float32) + b3_ref[...], 0.0)
  wo = wo_ref[...]
  dot = lax.dot_general(wo[:, :DIM], guv_ref[...], dn,
                        preferred_element_type=jnp.float32)
  dot = dot + lax.dot_general(wo[:, DIM:], h, dn,
                              preferred_element_type=jnp.float32)
  out_ref[...] = dot + bo_ref[0, 0]


def _tc_mlp(mu, mi, guv, w1, b1, w2, b2, w3, b3, wo, bo):
  nblk = 8
  blk = BATCH // nblk
  data_spec = pl.BlockSpec((DIM, blk), lambda i: (0, i))
  full = lambda shape: pl.BlockSpec(shape, lambda i: (0, 0))
  return pl.pallas_call(
      _tc_body,
      grid=(nblk,),
      in_specs=[
          data_spec, data_spec, data_spec,
          full(w1.shape), full(b1.shape),
          full(w2.shape), full(b2.shape),
          full(w3.shape), full(b3.shape),
          full(wo.shape), full(bo.shape),
      ],
      out_specs=pl.BlockSpec((1, blk), lambda i: (0, i)),
      out_shape=jax.ShapeDtypeStruct((1, BATCH), jnp.float32),
  )(mu, mi, guv, w1, b1, w2, b2, w3, b3, wo, bo)


@jax.jit
def kernel(user, item, GMF_U, GMF_I, MLP_U, MLP_I,
           W1, b1, W2, b2, W3, b3, Wo, bo):
  mu, mi, guv = _sc_gather(user, item, GMF_U.T, GMF_I.T, MLP_U.T, MLP_I.T)
  out = _tc_mlp(mu, mi, guv,
                W1, b1.reshape(-1, 1), W2, b2.reshape(-1, 1),
                W3, b3.reshape(-1, 1), Wo, bo.reshape(1, 1))
  return out.reshape(-1)
